# baseline (speedup 1.0000x reference)
import functools

import jax
import jax.numpy as jnp
from jax import lax
from jax.experimental import pallas as pl
from jax.experimental.pallas import tpu as pltpu
from jax.experimental.pallas import tpu_sc as plsc

N_NODES = 10000
N_EDGES = 320000
D_IN = 128
D_OUT = 256
D_HALF = 128

NC = 2
NS = 16
L = 16

NPAD = 10240
EB = 128
NCHUNK = N_EDGES // EB
BQ = 8
NBQ = NCHUNK // BQ
KFULL = NBQ // NS
BREM = NBQ - KFULL * NS
CREM = NCHUNK - NBQ * BQ
HB = NBQ // NC
KF2 = HB // NS
BR2 = HB - KF2 * NS
ROWS_PER_TILE = NPAD // NS
FB = 80
ZR = 32

_MESH = plsc.VectorSubcoreMesh(
    core_axis_name="c", subcore_axis_name="s", num_cores=NC, num_subcores=NS)


def _fill(ref, rows, cols, value):
    v = jnp.full((L,), value, jnp.float32)

    def row(r, _):
        for j in range(cols // L):
            ref[r, pl.ds(j * L, L)] = v
        return 0

    lax.fori_loop(0, rows, row, 0)


def _deg_body(dst_hbm, deg_hbm, deg_sh, idx_b, ones_v, buf_v):
    c = lax.axis_index("c")
    s = lax.axis_index("s")

    _fill(buf_v, 1, 320, 0.0)
    for k in range(ROWS_PER_TILE // 320):
        pltpu.sync_copy(buf_v.at[0], deg_sh.at[pl.ds(s * ROWS_PER_TILE + k * 320, 320)])
    _fill(ones_v, 1, EB, 1.0)
    plsc.subcore_barrier()

    def batch(t, nq):
        row = pl.multiple_of(t * BQ, 8)
        pltpu.sync_copy(dst_hbm.at[pl.ds(row, nq)], idx_b.at[pl.ds(0, nq)])
        for q in range(nq):
            pltpu.sync_copy(ones_v.at[0], deg_sh.at[idx_b.at[q]], add=True)

    def round_(k, _):
        batch(c * HB + k * NS + s, BQ)
        return 0

    lax.fori_loop(0, KF2, round_, 0)

    @pl.when(s < BR2)
    def _():
        batch(c * HB + KF2 * NS + s, BQ)

    @pl.when((c == 1) & (s == BR2))
    def _():
        batch(NBQ, CREM)

    plsc.subcore_barrier()

    for k in range(ROWS_PER_TILE // 320):
        me0 = pl.multiple_of(s * ROWS_PER_TILE + k * 320, 8)
        pltpu.sync_copy(deg_sh.at[pl.ds(me0, 320)], buf_v.at[0])
        pltpu.sync_copy(buf_v.at[0], deg_hbm.at[pl.ds(c * NPAD + me0, 320)])


_deg_kernel = functools.partial(
    pl.kernel,
    out_type=jax.ShapeDtypeStruct((NC * NPAD,), jnp.float32),
    mesh=_MESH,
    scratch_types=[
        pltpu.VMEM_SHARED((NPAD,), jnp.float32),
        pltpu.VMEM((BQ, EB), jnp.int32),
        pltpu.VMEM((1, EB), jnp.float32),
        pltpu.VMEM((1, 320), jnp.float32),
    ],
)(_deg_body)


def _mm_body(x_ref, w_ref, p0_ref, p1_ref, xs_ref, dis_ref):
    dis = lax.rsqrt(p0_ref[...] + p1_ref[...] + 1.0)
    xw = lax.dot_general(
        x_ref[...], w_ref[...],
        dimension_numbers=(((1,), (1,)), ((), ())),
        precision=lax.Precision.HIGHEST,
        preferred_element_type=jnp.float32)
    xs_ref[...] = xw * dis
    dis_ref[...] = dis


_MBLK = 256


def _matmul_scale(x_pad, W, deg2):
    nblk = NPAD // _MBLK
    return pl.pallas_call(
        _mm_body,
        grid=(NC, nblk),
        in_specs=[
            pl.BlockSpec((_MBLK, D_IN), lambda j, i: (i, 0)),
            pl.BlockSpec((D_HALF, D_IN), lambda j, i: (j, 0)),
            pl.BlockSpec((_MBLK, 1), lambda j, i: (i, 0)),
            pl.BlockSpec((_MBLK, 1), lambda j, i: (i + nblk, 0)),
        ],
        out_specs=[
            pl.BlockSpec((_MBLK, D_HALF), lambda j, i: (j * nblk + i, 0)),
            pl.BlockSpec((_MBLK, 1), lambda j, i: (i, 0)),
        ],
        out_shape=[
            jax.ShapeDtypeStruct((NC * NPAD, D_HALF), jnp.float32),
            jax.ShapeDtypeStruct((NPAD, 1), jnp.float32),
        ],
    )(x_pad, W, deg2, deg2)


def _edge_body(src_hbm, dst_hbm, xs_hbm, dis_hbm, b_hbm, out_hbm,
               acc_sh, src_b, dst_b, rows_v, accb_v,
               dis_v, b_v, gsem0, gsem1, ssem0, ssem1):
    c = lax.axis_index("c")
    s = lax.axis_index("s")
    off = c * NPAD

    for k in range(ROWS_PER_TILE // FB):
        base = pl.multiple_of(s * ROWS_PER_TILE + k * FB, 8)
        pltpu.sync_copy(xs_hbm.at[pl.ds(off + base, FB)], accb_v)
        pltpu.sync_copy(accb_v, acc_sh.at[pl.ds(base, FB)])
    plsc.subcore_barrier()

    gsems = (gsem0, gsem1)
    ssems = (ssem0, ssem1)

    def gstart(q, p):
        pltpu.make_async_copy(
            xs_hbm.at[src_b.at[q]], rows_v.at[p], gsems[p]).start()

    def gwait(q, p):
        pltpu.make_async_copy(
            xs_hbm.at[src_b.at[q]], rows_v.at[p], gsems[p]).wait()

    def sstart(q, p):
        pltpu.async_copy(
            rows_v.at[p], acc_sh.at[dst_b.at[q]], ssems[p], add=True)

    def swait(q, p):
        pltpu.make_async_copy(
            rows_v.at[p], acc_sh.at[dst_b.at[q]], ssems[p]).wait()

    def batch(t, nq):
        row = pl.multiple_of(t * BQ, 8)
        pltpu.sync_copy(src_hbm.at[pl.ds(row, nq)], src_b.at[pl.ds(0, nq)])
        pltpu.sync_copy(dst_hbm.at[pl.ds(row, nq)], dst_b.at[pl.ds(0, nq)])

        def addoff(q, _):
            for j in range(EB // L):
                sl = pl.ds(j * L, L)
                src_b[q, sl] = src_b[q, sl] + off
            return 0

        lax.fori_loop(0, nq, addoff, 0)

        gstart(0, 0)
        for q in range(nq):
            p = q % 2
            gwait(q, p)
            sstart(q, p)
            if q + 1 < nq:
                if q >= 1:
                    swait(q - 1, (q + 1) % 2)
                gstart(q + 1, (q + 1) % 2)
        swait(nq - 2, nq % 2)
        swait(nq - 1, (nq - 1) % 2)

    def round_(k, _):
        batch(k * NS + s, BQ)
        return 0

    lax.fori_loop(0, KFULL, round_, 0)

    @pl.when(s < BREM)
    def _():
        batch(KFULL * NS + s, BQ)

    @pl.when(s == BREM)
    def _():
        batch(NBQ, CREM)

    plsc.subcore_barrier()

    pltpu.sync_copy(b_hbm.at[pl.ds(c * D_HALF, D_HALF)], b_v.at[0])

    for k in range(ROWS_PER_TILE // FB):
        base = pl.multiple_of(s * ROWS_PER_TILE + k * FB, 8)
        pltpu.sync_copy(acc_sh.at[pl.ds(base, FB)], accb_v)
        pltpu.sync_copy(dis_hbm.at[pl.ds(base, FB)], dis_v.at[0])

        def grp(g, _):
            dvec = dis_v[0, pl.ds(g * L, L)]
            for rr in range(L):
                r = g * L + rr
                d = dvec[rr]

                def col(j, _):
                    sl = pl.ds(j * L, L)
                    v = accb_v[r, sl] * d + b_v[0, sl]
                    accb_v[r, sl] = jnp.maximum(v, jnp.float32(0.0))
                    return 0

                lax.fori_loop(0, D_HALF // L, col, 0)
            return 0

        lax.fori_loop(0, FB // L, grp, 0)
        pltpu.sync_copy(
            accb_v, out_hbm.at[pl.ds(base, FB), pl.ds(c * D_HALF, D_HALF)])


_edge_kernel = functools.partial(
    pl.kernel,
    out_type=jax.ShapeDtypeStruct((NPAD, D_OUT), jnp.float32),
    mesh=_MESH,
    scratch_types=[
        pltpu.VMEM_SHARED((NPAD, D_HALF), jnp.float32),
        pltpu.VMEM((BQ, EB), jnp.int32),
        pltpu.VMEM((BQ, EB), jnp.int32),
        pltpu.VMEM((2, EB, D_HALF), jnp.float32),
        pltpu.VMEM((FB, D_HALF), jnp.float32),
        pltpu.VMEM((1, FB), jnp.float32),
        pltpu.VMEM((1, D_HALF), jnp.float32),
        pltpu.SemaphoreType.DMA,
        pltpu.SemaphoreType.DMA,
        pltpu.SemaphoreType.DMA,
        pltpu.SemaphoreType.DMA,
    ],
)(_edge_body)


def kernel(x, edge_index, W, b):
    src = edge_index[:, 0].astype(jnp.int32).reshape(NCHUNK, EB)
    dst = edge_index[:, 1].astype(jnp.int32).reshape(NCHUNK, EB)

    deg = _deg_kernel(dst)

    x_pad = jnp.zeros((NPAD, D_IN), jnp.float32).at[:N_NODES].set(x)
    xs, dis = _matmul_scale(x_pad, W, deg.reshape(NC * NPAD, 1))

    out_pad = _edge_kernel(src, dst, xs, dis.reshape(NPAD), b)
    return out_pad[:N_NODES]


# speedup vs baseline: 1.0688x; 1.0688x over previous
import functools

import jax
import jax.numpy as jnp
from jax import lax
from jax.experimental import pallas as pl
from jax.experimental.pallas import tpu as pltpu
from jax.experimental.pallas import tpu_sc as plsc

N_NODES = 10000
N_EDGES = 320000
D_IN = 128
D_OUT = 256
D_HALF = 128

NC = 2
NS = 16
L = 16

NPAD = 10240
EB = 128
NCHUNK = N_EDGES // EB
BQ = 8
NBQ = NCHUNK // BQ
KFULL = NBQ // NS
BREM = NBQ - KFULL * NS
CREM = NCHUNK - NBQ * BQ
HB = NBQ // NC
KF2 = HB // NS
BR2 = HB - KF2 * NS
ROWS_PER_TILE = NPAD // NS
FB = 80
ZR = 32

_MESH = plsc.VectorSubcoreMesh(
    core_axis_name="c", subcore_axis_name="s", num_cores=NC, num_subcores=NS)


def _fill(ref, rows, cols, value):
    v = jnp.full((L,), value, jnp.float32)

    def row(r, _):
        for j in range(cols // L):
            ref[r, pl.ds(j * L, L)] = v
        return 0

    lax.fori_loop(0, rows, row, 0)


def _deg_body(dst_hbm, deg_hbm, deg_sh, idx_b, ones_v, buf_v):
    c = lax.axis_index("c")
    s = lax.axis_index("s")

    _fill(buf_v, 1, 320, 0.0)
    for k in range(ROWS_PER_TILE // 320):
        pltpu.sync_copy(buf_v.at[0], deg_sh.at[pl.ds(s * ROWS_PER_TILE + k * 320, 320)])
    _fill(ones_v, 1, EB, 1.0)
    plsc.subcore_barrier()

    def batch(t, nq):
        row = pl.multiple_of(t * BQ, 8)
        pltpu.sync_copy(dst_hbm.at[pl.ds(row, nq)], idx_b.at[pl.ds(0, nq)])
        for q in range(nq):
            pltpu.sync_copy(ones_v.at[0], deg_sh.at[idx_b.at[q]], add=True)

    def round_(k, _):
        batch(c * HB + k * NS + s, BQ)
        return 0

    lax.fori_loop(0, KF2, round_, 0)

    @pl.when(s < BR2)
    def _():
        batch(c * HB + KF2 * NS + s, BQ)

    @pl.when((c == 1) & (s == BR2))
    def _():
        batch(NBQ, CREM)

    plsc.subcore_barrier()

    for k in range(ROWS_PER_TILE // 320):
        me0 = pl.multiple_of(s * ROWS_PER_TILE + k * 320, 8)
        pltpu.sync_copy(deg_sh.at[pl.ds(me0, 320)], buf_v.at[0])
        pltpu.sync_copy(buf_v.at[0], deg_hbm.at[pl.ds(c * NPAD + me0, 320)])


_deg_kernel = functools.partial(
    pl.kernel,
    out_type=jax.ShapeDtypeStruct((NC * NPAD,), jnp.float32),
    mesh=_MESH,
    scratch_types=[
        pltpu.VMEM_SHARED((NPAD,), jnp.float32),
        pltpu.VMEM((BQ, EB), jnp.int32),
        pltpu.VMEM((1, EB), jnp.float32),
        pltpu.VMEM((1, 320), jnp.float32),
    ],
)(_deg_body)


def _mm_body(x_ref, w_ref, p0_ref, p1_ref, xs_ref, dis_ref):
    dis = lax.rsqrt(p0_ref[...] + p1_ref[...] + 1.0)
    xw = lax.dot_general(
        x_ref[...], w_ref[...],
        dimension_numbers=(((1,), (1,)), ((), ())),
        precision=lax.Precision.HIGHEST,
        preferred_element_type=jnp.float32)
    xs_ref[...] = xw * dis
    dis_ref[...] = dis


_MBLK = 256


def _matmul_scale(x_pad, W, deg2):
    nblk = NPAD // _MBLK
    return pl.pallas_call(
        _mm_body,
        grid=(NC, nblk),
        in_specs=[
            pl.BlockSpec((_MBLK, D_IN), lambda j, i: (i, 0)),
            pl.BlockSpec((D_HALF, D_IN), lambda j, i: (j, 0)),
            pl.BlockSpec((_MBLK, 1), lambda j, i: (i, 0)),
            pl.BlockSpec((_MBLK, 1), lambda j, i: (i + nblk, 0)),
        ],
        out_specs=[
            pl.BlockSpec((_MBLK, D_HALF), lambda j, i: (j * nblk + i, 0)),
            pl.BlockSpec((_MBLK, 1), lambda j, i: (i, 0)),
        ],
        out_shape=[
            jax.ShapeDtypeStruct((NC * NPAD, D_HALF), jnp.float32),
            jax.ShapeDtypeStruct((NPAD, 1), jnp.float32),
        ],
    )(x_pad, W, deg2, deg2)


def _edge_body(src_hbm, dst_hbm, xs_hbm, dis_hbm, b_hbm, out_hbm,
               acc_sh, src_b, dst_b, rows_v, accb_v,
               dis_v, b_v, gsem0, gsem1, ssem0, ssem1):
    c = lax.axis_index("c")
    s = lax.axis_index("s")
    off = c * NPAD

    for k in range(ROWS_PER_TILE // FB):
        base = pl.multiple_of(s * ROWS_PER_TILE + k * FB, 8)
        pltpu.sync_copy(xs_hbm.at[pl.ds(off + base, FB)], accb_v)
        pltpu.sync_copy(accb_v, acc_sh.at[pl.ds(base, FB)])
    plsc.subcore_barrier()

    gsems = (gsem0, gsem1)
    ssems = (ssem0, ssem1)

    def gstart(q, p):
        pltpu.make_async_copy(
            xs_hbm.at[src_b.at[q]], rows_v.at[p], gsems[p]).start()

    def gwait(q, p):
        pltpu.make_async_copy(
            xs_hbm.at[src_b.at[q]], rows_v.at[p], gsems[p]).wait()

    def sstart(q, p):
        pltpu.async_copy(
            rows_v.at[p], acc_sh.at[dst_b.at[q]], ssems[p], add=True)

    def swait(q, p):
        pltpu.make_async_copy(
            rows_v.at[p], acc_sh.at[dst_b.at[q]], ssems[p]).wait()

    def batch(t, nq):
        row = pl.multiple_of(t * BQ, 8)
        pltpu.sync_copy(src_hbm.at[pl.ds(row, nq)], src_b.at[pl.ds(0, nq)])
        pltpu.sync_copy(dst_hbm.at[pl.ds(row, nq)], dst_b.at[pl.ds(0, nq)])

        def addoff(q, _):
            for j in range(EB // L):
                sl = pl.ds(j * L, L)
                src_b[q, sl] = src_b[q, sl] + off
            return 0

        lax.fori_loop(0, nq, addoff, 0)

        gstart(0, 0)
        for q in range(nq):
            p = q % 2
            if q + 1 < nq:
                if q >= 1:
                    swait(q - 1, (q + 1) % 2)
                gstart(q + 1, (q + 1) % 2)
            gwait(q, p)
            sstart(q, p)
        swait(nq - 2, nq % 2)
        swait(nq - 1, (nq - 1) % 2)

    def round_(k, _):
        batch(k * NS + s, BQ)
        return 0

    lax.fori_loop(0, KFULL, round_, 0)

    @pl.when(s < BREM)
    def _():
        batch(KFULL * NS + s, BQ)

    @pl.when(s == BREM)
    def _():
        batch(NBQ, CREM)

    plsc.subcore_barrier()

    pltpu.sync_copy(b_hbm.at[pl.ds(c * D_HALF, D_HALF)], b_v.at[0])

    for k in range(ROWS_PER_TILE // FB):
        base = pl.multiple_of(s * ROWS_PER_TILE + k * FB, 8)
        pltpu.sync_copy(acc_sh.at[pl.ds(base, FB)], accb_v)
        pltpu.sync_copy(dis_hbm.at[pl.ds(base, FB)], dis_v.at[0])

        def grp(g, _):
            dvec = dis_v[0, pl.ds(g * L, L)]
            for rr in range(L):
                r = g * L + rr
                d = dvec[rr]

                def col(j, _):
                    sl = pl.ds(j * L, L)
                    v = accb_v[r, sl] * d + b_v[0, sl]
                    accb_v[r, sl] = jnp.maximum(v, jnp.float32(0.0))
                    return 0

                lax.fori_loop(0, D_HALF // L, col, 0)
            return 0

        lax.fori_loop(0, FB // L, grp, 0)
        pltpu.sync_copy(
            accb_v, out_hbm.at[pl.ds(base, FB), pl.ds(c * D_HALF, D_HALF)])


_edge_kernel = functools.partial(
    pl.kernel,
    out_type=jax.ShapeDtypeStruct((NPAD, D_OUT), jnp.float32),
    mesh=_MESH,
    scratch_types=[
        pltpu.VMEM_SHARED((NPAD, D_HALF), jnp.float32),
        pltpu.VMEM((BQ, EB), jnp.int32),
        pltpu.VMEM((BQ, EB), jnp.int32),
        pltpu.VMEM((2, EB, D_HALF), jnp.float32),
        pltpu.VMEM((FB, D_HALF), jnp.float32),
        pltpu.VMEM((1, FB), jnp.float32),
        pltpu.VMEM((1, D_HALF), jnp.float32),
        pltpu.SemaphoreType.DMA,
        pltpu.SemaphoreType.DMA,
        pltpu.SemaphoreType.DMA,
        pltpu.SemaphoreType.DMA,
    ],
)(_edge_body)


def kernel(x, edge_index, W, b):
    src = edge_index[:, 0].astype(jnp.int32).reshape(NCHUNK, EB)
    dst = edge_index[:, 1].astype(jnp.int32).reshape(NCHUNK, EB)

    deg = _deg_kernel(dst)

    x_pad = jnp.zeros((NPAD, D_IN), jnp.float32).at[:N_NODES].set(x)
    xs, dis = _matmul_scale(x_pad, W, deg.reshape(NC * NPAD, 1))

    out_pad = _edge_kernel(src, dst, xs, dis.reshape(NPAD), b)
    return out_pad[:N_NODES]


# speedup vs baseline: 1.1006x; 1.0298x over previous
import functools

import jax
import jax.numpy as jnp
from jax import lax
from jax.experimental import pallas as pl
from jax.experimental.pallas import tpu as pltpu
from jax.experimental.pallas import tpu_sc as plsc

N_NODES = 10000
N_EDGES = 320000
D_IN = 128
D_OUT = 256
D_HALF = 128

NC = 2
NS = 16
L = 16

NPAD = 10240
EB = 128
NCHUNK = N_EDGES // EB
BQ = 8
NBQ = NCHUNK // BQ
KFULL = NBQ // NS
BREM = NBQ - KFULL * NS
CREM = NCHUNK - NBQ * BQ
HB = NBQ // NC
KF2 = HB // NS
BR2 = HB - KF2 * NS
ROWS_PER_TILE = NPAD // NS
FB = 80
ZR = 32

_MESH = plsc.VectorSubcoreMesh(
    core_axis_name="c", subcore_axis_name="s", num_cores=NC, num_subcores=NS)


def _fill(ref, rows, cols, value):
    v = jnp.full((L,), value, jnp.float32)

    def row(r, _):
        for j in range(cols // L):
            ref[r, pl.ds(j * L, L)] = v
        return 0

    lax.fori_loop(0, rows, row, 0)


def _deg_body(dst_hbm, deg_hbm, deg_sh, idx_b, ones_v, buf_v):
    c = lax.axis_index("c")
    s = lax.axis_index("s")

    _fill(buf_v, 1, 320, 0.0)
    for k in range(ROWS_PER_TILE // 320):
        pltpu.sync_copy(buf_v.at[0], deg_sh.at[pl.ds(s * ROWS_PER_TILE + k * 320, 320)])
    _fill(ones_v, 1, EB, 1.0)
    plsc.subcore_barrier()

    def batch(t, nq):
        row = pl.multiple_of(t * BQ, 8)
        pltpu.sync_copy(dst_hbm.at[pl.ds(row, nq)], idx_b.at[pl.ds(0, nq)])
        for q in range(nq):
            pltpu.sync_copy(ones_v.at[0], deg_sh.at[idx_b.at[q]], add=True)

    def round_(k, _):
        batch(c * HB + k * NS + s, BQ)
        return 0

    lax.fori_loop(0, KF2, round_, 0)

    @pl.when(s < BR2)
    def _():
        batch(c * HB + KF2 * NS + s, BQ)

    @pl.when((c == 1) & (s == BR2))
    def _():
        batch(NBQ, CREM)

    plsc.subcore_barrier()

    for k in range(ROWS_PER_TILE // 320):
        me0 = pl.multiple_of(s * ROWS_PER_TILE + k * 320, 8)
        pltpu.sync_copy(deg_sh.at[pl.ds(me0, 320)], buf_v.at[0])
        pltpu.sync_copy(buf_v.at[0], deg_hbm.at[pl.ds(c * NPAD + me0, 320)])


_deg_kernel = functools.partial(
    pl.kernel,
    out_type=jax.ShapeDtypeStruct((NC * NPAD,), jnp.float32),
    mesh=_MESH,
    scratch_types=[
        pltpu.VMEM_SHARED((NPAD,), jnp.float32),
        pltpu.VMEM((BQ, EB), jnp.int32),
        pltpu.VMEM((1, EB), jnp.float32),
        pltpu.VMEM((1, 320), jnp.float32),
    ],
)(_deg_body)


def _mm_body(x_ref, w_ref, p0_ref, p1_ref, xs_ref, dis_ref):
    dis = lax.rsqrt(p0_ref[...] + p1_ref[...] + 1.0)
    xw = lax.dot_general(
        x_ref[...], w_ref[...],
        dimension_numbers=(((1,), (1,)), ((), ())),
        precision=lax.Precision.HIGHEST,
        preferred_element_type=jnp.float32)
    xs_ref[...] = xw * dis
    dis_ref[...] = dis


_MBLK = 256


def _matmul_scale(x_pad, W, deg2):
    nblk = NPAD // _MBLK
    return pl.pallas_call(
        _mm_body,
        grid=(NC, nblk),
        in_specs=[
            pl.BlockSpec((_MBLK, D_IN), lambda j, i: (i, 0)),
            pl.BlockSpec((D_HALF, D_IN), lambda j, i: (j, 0)),
            pl.BlockSpec((_MBLK, 1), lambda j, i: (i, 0)),
            pl.BlockSpec((_MBLK, 1), lambda j, i: (i + nblk, 0)),
        ],
        out_specs=[
            pl.BlockSpec((_MBLK, D_HALF), lambda j, i: (j * nblk + i, 0)),
            pl.BlockSpec((_MBLK, 1), lambda j, i: (i, 0)),
        ],
        out_shape=[
            jax.ShapeDtypeStruct((NC * NPAD, D_HALF), jnp.float32),
            jax.ShapeDtypeStruct((NPAD, 1), jnp.float32),
        ],
    )(x_pad, W, deg2, deg2)


def _edge_body(src_hbm, dst_hbm, xs_hbm, dis_hbm, b_hbm, out_hbm,
               acc_sh, src_b, dst_b, rows_v, accb_v,
               dis_v, b_v, gsem0, gsem1, ssem0, ssem1, isem_s, isem_d):
    c = lax.axis_index("c")
    s = lax.axis_index("s")
    off = c * NPAD

    for k in range(ROWS_PER_TILE // FB):
        base = pl.multiple_of(s * ROWS_PER_TILE + k * FB, 8)
        pltpu.sync_copy(xs_hbm.at[pl.ds(off + base, FB)], accb_v)
        pltpu.sync_copy(accb_v, acc_sh.at[pl.ds(base, FB)])
    plsc.subcore_barrier()

    gsems = (gsem0, gsem1)
    ssems = (ssem0, ssem1)

    def gstart(q, p):
        pltpu.make_async_copy(
            xs_hbm.at[src_b.at[q]], rows_v.at[p], gsems[p]).start()

    def gwait(q, p):
        pltpu.make_async_copy(
            xs_hbm.at[src_b.at[q]], rows_v.at[p], gsems[p]).wait()

    def sstart(q, p):
        pltpu.async_copy(
            rows_v.at[p], acc_sh.at[dst_b.at[q]], ssems[p], add=True)

    def swait(q, p):
        pltpu.make_async_copy(
            rows_v.at[p], acc_sh.at[dst_b.at[q]], ssems[p]).wait()

    def batch(t, nq):
        row = pl.multiple_of(t * BQ, 8)
        scp = pltpu.make_async_copy(
            src_hbm.at[pl.ds(row, nq)], src_b.at[pl.ds(0, nq)], isem_s)
        dcp = pltpu.make_async_copy(
            dst_hbm.at[pl.ds(row, nq)], dst_b.at[pl.ds(0, nq)], isem_d)
        scp.start()
        dcp.start()
        scp.wait()

        def addoff(q, _):
            for j in range(EB // L):
                sl = pl.ds(j * L, L)
                src_b[q, sl] = src_b[q, sl] + off
            return 0

        lax.fori_loop(0, nq, addoff, 0)

        gstart(0, 0)
        dcp.wait()
        for q in range(nq):
            p = q % 2
            if q + 1 < nq:
                if q >= 1:
                    swait(q - 1, (q + 1) % 2)
                gstart(q + 1, (q + 1) % 2)
            gwait(q, p)
            sstart(q, p)
        swait(nq - 2, nq % 2)
        swait(nq - 1, (nq - 1) % 2)

    def round_(k, _):
        batch(k * NS + s, BQ)
        return 0

    lax.fori_loop(0, KFULL, round_, 0)

    @pl.when(s < BREM)
    def _():
        batch(KFULL * NS + s, BQ)

    @pl.when(s == BREM)
    def _():
        batch(NBQ, CREM)

    plsc.subcore_barrier()

    pltpu.sync_copy(b_hbm.at[pl.ds(c * D_HALF, D_HALF)], b_v.at[0])

    for k in range(ROWS_PER_TILE // FB):
        base = pl.multiple_of(s * ROWS_PER_TILE + k * FB, 8)
        pltpu.sync_copy(acc_sh.at[pl.ds(base, FB)], accb_v)
        pltpu.sync_copy(dis_hbm.at[pl.ds(base, FB)], dis_v.at[0])

        def grp(g, _):
            dvec = dis_v[0, pl.ds(g * L, L)]
            for rr in range(L):
                r = g * L + rr
                d = dvec[rr]

                def col(j, _):
                    sl = pl.ds(j * L, L)
                    v = accb_v[r, sl] * d + b_v[0, sl]
                    accb_v[r, sl] = jnp.maximum(v, jnp.float32(0.0))
                    return 0

                lax.fori_loop(0, D_HALF // L, col, 0)
            return 0

        lax.fori_loop(0, FB // L, grp, 0)
        pltpu.sync_copy(
            accb_v, out_hbm.at[pl.ds(base, FB), pl.ds(c * D_HALF, D_HALF)])


_edge_kernel = functools.partial(
    pl.kernel,
    out_type=jax.ShapeDtypeStruct((NPAD, D_OUT), jnp.float32),
    mesh=_MESH,
    scratch_types=[
        pltpu.VMEM_SHARED((NPAD, D_HALF), jnp.float32),
        pltpu.VMEM((BQ, EB), jnp.int32),
        pltpu.VMEM((BQ, EB), jnp.int32),
        pltpu.VMEM((2, EB, D_HALF), jnp.float32),
        pltpu.VMEM((FB, D_HALF), jnp.float32),
        pltpu.VMEM((1, FB), jnp.float32),
        pltpu.VMEM((1, D_HALF), jnp.float32),
        pltpu.SemaphoreType.DMA,
        pltpu.SemaphoreType.DMA,
        pltpu.SemaphoreType.DMA,
        pltpu.SemaphoreType.DMA,
        pltpu.SemaphoreType.DMA,
        pltpu.SemaphoreType.DMA,
    ],
)(_edge_body)


def kernel(x, edge_index, W, b):
    src = edge_index[:, 0].astype(jnp.int32).reshape(NCHUNK, EB)
    dst = edge_index[:, 1].astype(jnp.int32).reshape(NCHUNK, EB)

    deg = _deg_kernel(dst)

    x_pad = jnp.zeros((NPAD, D_IN), jnp.float32).at[:N_NODES].set(x)
    xs, dis = _matmul_scale(x_pad, W, deg.reshape(NC * NPAD, 1))

    out_pad = _edge_kernel(src, dst, xs, dis.reshape(NPAD), b)
    return out_pad[:N_NODES]


# speedup vs baseline: 1.1225x; 1.0199x over previous
import functools

import jax
import jax.numpy as jnp
from jax import lax
from jax.experimental import pallas as pl
from jax.experimental.pallas import tpu as pltpu
from jax.experimental.pallas import tpu_sc as plsc

N_NODES = 10000
N_EDGES = 320000
D_IN = 128
D_OUT = 256
D_HALF = 128

NC = 2
NS = 16
L = 16

NPAD = 10240
EB = 128
NCHUNK = N_EDGES // EB
BQ = 8
NBQ = NCHUNK // BQ
KFULL = NBQ // NS
BREM = NBQ - KFULL * NS
CREM = NCHUNK - NBQ * BQ
HB = NBQ // NC
KF2 = HB // NS
BR2 = HB - KF2 * NS
ROWS_PER_TILE = NPAD // NS
FB = 80
ZR = 32

_MESH = plsc.VectorSubcoreMesh(
    core_axis_name="c", subcore_axis_name="s", num_cores=NC, num_subcores=NS)


def _fill(ref, rows, cols, value):
    v = jnp.full((L,), value, jnp.float32)

    def row(r, _):
        for j in range(cols // L):
            ref[r, pl.ds(j * L, L)] = v
        return 0

    lax.fori_loop(0, rows, row, 0)


def _deg_body(dst_hbm, deg_hbm, deg_sh, idx_b, ones_v, buf_v, hsem0, hsem1):
    c = lax.axis_index("c")
    s = lax.axis_index("s")

    _fill(buf_v, 1, 320, 0.0)
    for k in range(ROWS_PER_TILE // 320):
        pltpu.sync_copy(buf_v.at[0], deg_sh.at[pl.ds(s * ROWS_PER_TILE + k * 320, 320)])
    _fill(ones_v, 1, EB, 1.0)
    plsc.subcore_barrier()

    hsems = (hsem0, hsem1)

    def batch(t, nq):
        row = pl.multiple_of(t * BQ, 8)
        pltpu.sync_copy(dst_hbm.at[pl.ds(row, nq)], idx_b.at[pl.ds(0, nq)])
        for q in range(nq):
            if q >= 2:
                pltpu.make_async_copy(
                    ones_v.at[0], deg_sh.at[idx_b.at[q - 2]], hsems[q % 2]).wait()
            pltpu.async_copy(
                ones_v.at[0], deg_sh.at[idx_b.at[q]], hsems[q % 2], add=True)
        for q in range(max(0, nq - 2), nq):
            pltpu.make_async_copy(
                ones_v.at[0], deg_sh.at[idx_b.at[q]], hsems[q % 2]).wait()

    def round_(k, _):
        batch(c * HB + k * NS + s, BQ)
        return 0

    lax.fori_loop(0, KF2, round_, 0)

    @pl.when(s < BR2)
    def _():
        batch(c * HB + KF2 * NS + s, BQ)

    @pl.when((c == 1) & (s == BR2))
    def _():
        batch(NBQ, CREM)

    plsc.subcore_barrier()

    for k in range(ROWS_PER_TILE // 320):
        me0 = pl.multiple_of(s * ROWS_PER_TILE + k * 320, 8)
        pltpu.sync_copy(deg_sh.at[pl.ds(me0, 320)], buf_v.at[0])
        pltpu.sync_copy(buf_v.at[0], deg_hbm.at[pl.ds(c * NPAD + me0, 320)])


_deg_kernel = functools.partial(
    pl.kernel,
    out_type=jax.ShapeDtypeStruct((NC * NPAD,), jnp.float32),
    mesh=_MESH,
    scratch_types=[
        pltpu.VMEM_SHARED((NPAD,), jnp.float32),
        pltpu.VMEM((BQ, EB), jnp.int32),
        pltpu.VMEM((1, EB), jnp.float32),
        pltpu.VMEM((1, 320), jnp.float32),
        pltpu.SemaphoreType.DMA,
        pltpu.SemaphoreType.DMA,
    ],
)(_deg_body)


def _mm_body(x_ref, w_ref, p0_ref, p1_ref, xs_ref, dis_ref):
    dis = lax.rsqrt(p0_ref[...] + p1_ref[...] + 1.0)
    xw = lax.dot_general(
        x_ref[...], w_ref[...],
        dimension_numbers=(((1,), (1,)), ((), ())),
        precision=lax.Precision.HIGHEST,
        preferred_element_type=jnp.float32)
    xs_ref[...] = xw * dis
    dis_ref[...] = dis


_MBLK = 256


def _matmul_scale(x_pad, W, deg2):
    nblk = NPAD // _MBLK
    return pl.pallas_call(
        _mm_body,
        grid=(NC, nblk),
        in_specs=[
            pl.BlockSpec((_MBLK, D_IN), lambda j, i: (i, 0)),
            pl.BlockSpec((D_HALF, D_IN), lambda j, i: (j, 0)),
            pl.BlockSpec((_MBLK, 1), lambda j, i: (i, 0)),
            pl.BlockSpec((_MBLK, 1), lambda j, i: (i + nblk, 0)),
        ],
        out_specs=[
            pl.BlockSpec((_MBLK, D_HALF), lambda j, i: (j * nblk + i, 0)),
            pl.BlockSpec((_MBLK, 1), lambda j, i: (i, 0)),
        ],
        out_shape=[
            jax.ShapeDtypeStruct((NC * NPAD, D_HALF), jnp.float32),
            jax.ShapeDtypeStruct((NPAD, 1), jnp.float32),
        ],
    )(x_pad, W, deg2, deg2)


def _edge_body(src_hbm, dst_hbm, xs_hbm, dis_hbm, b_hbm, out_hbm,
               acc_sh, src_b, dst_b, rows_v, accb_v,
               dis_v, b_v, gsem0, gsem1, ssem0, ssem1, isem_s, isem_d):
    c = lax.axis_index("c")
    s = lax.axis_index("s")
    off = c * NPAD

    for k in range(ROWS_PER_TILE // FB):
        base = pl.multiple_of(s * ROWS_PER_TILE + k * FB, 8)
        pltpu.sync_copy(xs_hbm.at[pl.ds(off + base, FB)], accb_v)
        pltpu.sync_copy(accb_v, acc_sh.at[pl.ds(base, FB)])
    plsc.subcore_barrier()

    gsems = (gsem0, gsem1)
    ssems = (ssem0, ssem1)

    def gstart(q, p):
        pltpu.make_async_copy(
            xs_hbm.at[src_b.at[q]], rows_v.at[p], gsems[p]).start()

    def gwait(q, p):
        pltpu.make_async_copy(
            xs_hbm.at[src_b.at[q]], rows_v.at[p], gsems[p]).wait()

    def sstart(q, p):
        pltpu.async_copy(
            rows_v.at[p], acc_sh.at[dst_b.at[q]], ssems[p], add=True)

    def swait(q, p):
        pltpu.make_async_copy(
            rows_v.at[p], acc_sh.at[dst_b.at[q]], ssems[p]).wait()

    def batch(t, nq):
        row = pl.multiple_of(t * BQ, 8)
        scp = pltpu.make_async_copy(
            src_hbm.at[pl.ds(row, nq)], src_b.at[pl.ds(0, nq)], isem_s)
        dcp = pltpu.make_async_copy(
            dst_hbm.at[pl.ds(row, nq)], dst_b.at[pl.ds(0, nq)], isem_d)
        scp.start()
        dcp.start()
        scp.wait()

        def addoff(q, _):
            for j in range(EB // L):
                sl = pl.ds(j * L, L)
                src_b[q, sl] = src_b[q, sl] + off
            return 0

        lax.fori_loop(0, nq, addoff, 0)

        gstart(0, 0)
        dcp.wait()
        for q in range(nq):
            p = q % 2
            if q + 1 < nq:
                if q >= 1:
                    swait(q - 1, (q + 1) % 2)
                gstart(q + 1, (q + 1) % 2)
            gwait(q, p)
            sstart(q, p)
        swait(nq - 2, nq % 2)
        swait(nq - 1, (nq - 1) % 2)

    def round_(k, _):
        batch(k * NS + s, BQ)
        return 0

    lax.fori_loop(0, KFULL, round_, 0)

    @pl.when(s < BREM)
    def _():
        batch(KFULL * NS + s, BQ)

    @pl.when(s == BREM)
    def _():
        batch(NBQ, CREM)

    plsc.subcore_barrier()

    pltpu.sync_copy(b_hbm.at[pl.ds(c * D_HALF, D_HALF)], b_v.at[0])
    tile0 = pl.multiple_of(s * ROWS_PER_TILE, 8)
    pltpu.sync_copy(dis_hbm.at[pl.ds(tile0, ROWS_PER_TILE)], dis_v.at[0])

    for k in range(ROWS_PER_TILE // FB):
        base = pl.multiple_of(s * ROWS_PER_TILE + k * FB, 8)
        pltpu.sync_copy(acc_sh.at[pl.ds(base, FB)], accb_v)

        def grp(g, _):
            dvec = dis_v[0, pl.ds(k * FB + g * L, L)]
            for rr in range(L):
                r = g * L + rr
                d = dvec[rr]

                def col(j, _):
                    sl = pl.ds(j * L, L)
                    v = accb_v[r, sl] * d + b_v[0, sl]
                    accb_v[r, sl] = jnp.maximum(v, jnp.float32(0.0))
                    return 0

                lax.fori_loop(0, D_HALF // L, col, 0)
            return 0

        lax.fori_loop(0, FB // L, grp, 0)
        pltpu.sync_copy(
            accb_v, out_hbm.at[pl.ds(base, FB), pl.ds(c * D_HALF, D_HALF)])


_edge_kernel = functools.partial(
    pl.kernel,
    out_type=jax.ShapeDtypeStruct((NPAD, D_OUT), jnp.float32),
    mesh=_MESH,
    scratch_types=[
        pltpu.VMEM_SHARED((NPAD, D_HALF), jnp.float32),
        pltpu.VMEM((BQ, EB), jnp.int32),
        pltpu.VMEM((BQ, EB), jnp.int32),
        pltpu.VMEM((2, EB, D_HALF), jnp.float32),
        pltpu.VMEM((FB, D_HALF), jnp.float32),
        pltpu.VMEM((1, ROWS_PER_TILE), jnp.float32),
        pltpu.VMEM((1, D_HALF), jnp.float32),
        pltpu.SemaphoreType.DMA,
        pltpu.SemaphoreType.DMA,
        pltpu.SemaphoreType.DMA,
        pltpu.SemaphoreType.DMA,
        pltpu.SemaphoreType.DMA,
        pltpu.SemaphoreType.DMA,
    ],
)(_edge_body)


def kernel(x, edge_index, W, b):
    src = edge_index[:, 0].astype(jnp.int32).reshape(NCHUNK, EB)
    dst = edge_index[:, 1].astype(jnp.int32).reshape(NCHUNK, EB)

    deg = _deg_kernel(dst)

    x_pad = jnp.zeros((NPAD, D_IN), jnp.float32).at[:N_NODES].set(x)
    xs, dis = _matmul_scale(x_pad, W, deg.reshape(NC * NPAD, 1))

    out_pad = _edge_kernel(src, dst, xs, dis.reshape(NPAD), b)
    return out_pad[:N_NODES]
